# baseline (speedup 1.0000x reference)
import functools

import jax
import jax.numpy as jnp
from jax import lax
from jax.experimental import pallas as pl
from jax.experimental.pallas import tpu as pltpu
from jax.experimental.pallas import tpu_sc as plsc

_D = 16
_B = 16384
_H = 50
_C = 20
_BETA = 0.5

_HP = 56
_CP = 24
_OP = 32

_NC = 2
_NS = 16
_NW = _NC * _NS
_RPW = _B // _NW

_WH_LINES = (1000001 + 7) // 8 * 8 * 16 // 128 + 1
_CT_LINES = (1000001 + 3) // 4 * 4 * 32 // 128 + 1


def _sc_body(hg_hbm, hcol_hbm, cg_hbm, ccol_hbm, wh_hbm, wct_hbm, out_hbm,
             hg_v, hcol_v, cg_v, ccol_v, out_v,
             hb0, cb0, hb1, cb1,
             hs0, cs0, hs1, cs1):
    wid = lax.axis_index("s") * _NC + lax.axis_index("c")
    base = wid * _RPW

    pltpu.sync_copy(hg_hbm.at[pl.ds(base * _HP, _RPW * _HP)], hg_v)
    pltpu.sync_copy(hcol_hbm.at[pl.ds(base * _HP, _RPW * _HP)],
                    hcol_v.at[pl.ds(0, _RPW * _HP)])
    pltpu.sync_copy(cg_hbm.at[pl.ds(base * _CP, _RPW * _CP)], cg_v)
    pltpu.sync_copy(ccol_hbm.at[pl.ds(base * _CP, _RPW * _CP)],
                    ccol_v.at[pl.ds(0, _RPW * _CP)])

    hbufs = (hb0, hb1)
    cbufs = (cb0, cb1)
    hsems = (hs0, hs1)
    csems = (cs0, cs1)

    def issue(row, b):
        pltpu.async_copy(
            wh_hbm.at[hg_v.at[pl.ds(row * _HP, _H)]], hbufs[b], hsems[b])
        pltpu.async_copy(
            wct_hbm.at[cg_v.at[pl.ds(row * _CP, _C)]], cbufs[b], csems[b])

    def wait(row, b):
        pltpu.make_async_copy(
            wh_hbm.at[hg_v.at[pl.ds(row * _HP, _H)]], hbufs[b],
            hsems[b]).wait()
        pltpu.make_async_copy(
            wct_hbm.at[cg_v.at[pl.ds(row * _CP, _C)]], cbufs[b],
            csems[b]).wait()

    lanes = lax.iota(jnp.int32, _D)

    def compute(row, b):
        hb = hbufs[b]
        cb = cbufs[b]
        hcw = [hcol_v[pl.ds(row * _HP + 16 * k, 16)]
               for k in range((_H + 15) // 16)]
        ccw = [ccol_v[pl.ds(row * _CP + 16 * k, 16)]
               for k in range((_C + 15) // 16)]
        acc = None
        for h in range(_H):
            col = hcw[h // 16][h % 16]
            vec = hb[h, pl.ds(col, _D)]
            term = vec if h == 0 else vec * (_BETA ** h)
            acc = term if acc is None else acc + term
        cols = [ccw[c // 16][c % 16] for c in range(_C)]
        ctx = [cb[c, pl.ds(cols[c], _D)] for c in range(_C)]
        s = ctx[0]
        for c in range(1, _C):
            s = s + ctx[c]
        a = acc + s
        lo = jnp.zeros((_D,), jnp.float32)
        hi = jnp.zeros((_D,), jnp.float32)
        for c in range(_C):
            tgt = cb[c, pl.ds(cols[c] + _D, _D)]
            u = jnp.sum(tgt * (a - ctx[c]))
            if c < _D:
                lo = jnp.where(lanes == c, u, lo)
            else:
                hi = jnp.where(lanes == (c - _D), u, hi)
        out_v[pl.ds(row * _OP, _D)] = lo
        out_v[pl.ds(row * _OP + _D, _D)] = hi

    issue(0, 0)

    def body(i, carry):
        r = i * 2
        for b in range(2):
            row = r + b
            nxt = row + 1

            @pl.when(nxt < _RPW)
            def _():
                issue(nxt, 1 - b)

            wait(row, b)
            compute(row, b)
        return carry

    lax.fori_loop(0, _RPW // 2, body, 0, unroll=False)

    pltpu.sync_copy(out_v, out_hbm.at[pl.ds(base * _OP, _RPW * _OP)])


_sc_utilities = functools.partial(
    pl.kernel,
    out_type=jax.ShapeDtypeStruct((_B * _OP,), jnp.float32),
    mesh=plsc.VectorSubcoreMesh(core_axis_name="c", subcore_axis_name="s"),
    compiler_params=pltpu.CompilerParams(
        needs_layout_passes=False, use_tc_tiling_on_sc=False),
    scratch_types=[
        pltpu.VMEM((_RPW * _HP,), jnp.int32),
        pltpu.VMEM((_RPW * _HP + 16,), jnp.int32),
        pltpu.VMEM((_RPW * _CP,), jnp.int32),
        pltpu.VMEM((_RPW * _CP + 16,), jnp.int32),
        pltpu.VMEM((_RPW * _OP,), jnp.float32),
        pltpu.VMEM((_H, 128), jnp.float32),
        pltpu.VMEM((_C, 128), jnp.float32),
        pltpu.VMEM((_H, 128), jnp.float32),
        pltpu.VMEM((_C, 128), jnp.float32),
        pltpu.SemaphoreType.DMA,
        pltpu.SemaphoreType.DMA,
        pltpu.SemaphoreType.DMA,
        pltpu.SemaphoreType.DMA,
    ],
)(_sc_body)


def _softmax_body(u_ref, len_ref, o_ref):
    u = u_ref[...]
    ln = len_ref[...]
    col = lax.broadcasted_iota(jnp.int32, u.shape, 1)
    u = jnp.where((col >= ln) | (col >= _C), -jnp.inf, u)
    m = jnp.max(u, axis=1, keepdims=True)
    sh = u - m
    lse = jnp.log(jnp.sum(jnp.exp(sh), axis=1, keepdims=True))
    o_ref[...] = (sh - lse)[:, :_C]


_BLK = 2048


def _tc_logsoftmax(util, lens2d):
    return pl.pallas_call(
        _softmax_body,
        grid=(_B // _BLK,),
        in_specs=[
            pl.BlockSpec((_BLK, _OP), lambda i: (i, 0)),
            pl.BlockSpec((_BLK, 1), lambda i: (i, 0)),
        ],
        out_specs=pl.BlockSpec((_BLK, _C), lambda i: (i, 0)),
        out_shape=jax.ShapeDtypeStruct((_B, _C), jnp.float32),
    )(util, lens2d)


def kernel(histories, history_lengths, choice_sets, choice_set_lengths,
           Wh, Wc, Wt):
    del history_lengths
    hp = jnp.pad(histories, ((0, 0), (0, _HP - _H)))
    cp = jnp.pad(choice_sets, ((0, 0), (0, _CP - _C)))
    hg = (hp >> 3).reshape(-1)
    hcol = ((hp & 7) << 4).reshape(-1)
    cg = (cp >> 2).reshape(-1)
    ccol = ((cp & 3) << 5).reshape(-1)
    wh = jnp.pad(Wh, ((0, _WH_LINES * 8 - 1000001), (0, 0))).reshape(
        _WH_LINES, 128)
    wct = jnp.pad(jnp.concatenate([Wc, Wt], axis=1),
                  ((0, _CT_LINES * 4 - 1000001), (0, 0))).reshape(
        _CT_LINES, 128)
    util = _sc_utilities(hg, hcol, cg, ccol, wh, wct).reshape(_B, _OP)
    return _tc_logsoftmax(util, choice_set_lengths.reshape(_B, 1))
